# initial kernel scaffold (unmeasured)
import jax
import jax.numpy as jnp
from jax import lax
from jax.experimental import pallas as pl
from jax.experimental.pallas import tpu as pltpu


def kernel(
    x,
):
    def body(*refs):
        pass

    out_shape = jax.ShapeDtypeStruct(..., jnp.float32)
    return pl.pallas_call(body, out_shape=out_shape)(...)


# baseline (device time: 34702 ns/iter reference)
import jax
import jax.numpy as jnp
from jax import lax
from jax.experimental import pallas as pl
from jax.experimental.pallas import tpu as pltpu

N_GLOBAL = 2048


def kernel(x):
    m, n = x.shape
    BM = 512
    grid = m // BM

    def body(x_ref, out_ref, comm_ref, send_sem, recv_sem):
        i = pl.program_id(0)

        comm_ref[0, pl.ds(i * BM, BM), :] = jnp.sum(
            x_ref[...], axis=1, keepdims=True
        )

        @pl.when(i == grid - 1)
        def _():
            my_x = lax.axis_index("x")
            my_y = lax.axis_index("y")
            peer = (my_x, 1 - my_y)

            barrier_sem = pltpu.get_barrier_semaphore()
            pl.semaphore_signal(
                barrier_sem,
                inc=1,
                device_id=peer,
                device_id_type=pl.DeviceIdType.MESH,
            )
            pl.semaphore_wait(barrier_sem, 1)

            rdma = pltpu.make_async_remote_copy(
                src_ref=comm_ref.at[0],
                dst_ref=comm_ref.at[1],
                send_sem=send_sem,
                recv_sem=recv_sem,
                device_id=peer,
                device_id_type=pl.DeviceIdType.MESH,
            )
            rdma.start()
            rdma.wait()

            out_ref[...] = (comm_ref[0] + comm_ref[1]) * (1.0 / N_GLOBAL)

    return pl.pallas_call(
        body,
        grid=(grid,),
        in_specs=[pl.BlockSpec((BM, n), lambda i: (i, 0))],
        out_specs=pl.BlockSpec((m, 1), lambda i: (0, 0)),
        out_shape=jax.ShapeDtypeStruct((m, 1), jnp.float32),
        scratch_shapes=[
            pltpu.VMEM((2, m, 1), jnp.float32),
            pltpu.SemaphoreType.DMA,
            pltpu.SemaphoreType.DMA,
        ],
        compiler_params=pltpu.CompilerParams(collective_id=0),
    )(x)


# device time: 11574 ns/iter; 2.9983x vs baseline; 2.9983x over previous
import jax
import jax.numpy as jnp
from jax import lax
from jax.experimental import pallas as pl
from jax.experimental.pallas import tpu as pltpu

N_GLOBAL = 2048


def kernel(x):
    m, n = x.shape
    BM = 512
    grid = m // BM
    pm = m // 128
    pb = BM // 128

    def body(x_ref, out_ref, comm_ref, send_sem, recv_sem):
        i = pl.program_id(0)

        blk = jnp.sum(x_ref[...], axis=1)
        comm_ref[0, pl.ds(i * pb, pb), :] = blk.reshape(pb, 128)

        @pl.when(i == grid - 1)
        def _():
            my_x = lax.axis_index("x")
            my_y = lax.axis_index("y")
            peer = (my_x, 1 - my_y)

            barrier_sem = pltpu.get_barrier_semaphore()
            pl.semaphore_signal(
                barrier_sem,
                inc=1,
                device_id=peer,
                device_id_type=pl.DeviceIdType.MESH,
            )
            pl.semaphore_wait(barrier_sem, 1)

            rdma = pltpu.make_async_remote_copy(
                src_ref=comm_ref.at[0],
                dst_ref=comm_ref.at[1],
                send_sem=send_sem,
                recv_sem=recv_sem,
                device_id=peer,
                device_id_type=pl.DeviceIdType.MESH,
            )
            rdma.start()
            rdma.wait()

            out_ref[...] = (comm_ref[0] + comm_ref[1]) * (1.0 / N_GLOBAL)

    packed = pl.pallas_call(
        body,
        grid=(grid,),
        in_specs=[pl.BlockSpec((BM, n), lambda i: (i, 0))],
        out_specs=pl.BlockSpec((pm, 128), lambda i: (0, 0)),
        out_shape=jax.ShapeDtypeStruct((pm, 128), jnp.float32),
        scratch_shapes=[
            pltpu.VMEM((2, pm, 128), jnp.float32),
            pltpu.SemaphoreType.DMA,
            pltpu.SemaphoreType.DMA,
        ],
        compiler_params=pltpu.CompilerParams(collective_id=0),
    )(x)
    return packed.reshape(m, 1)


# device time: 11572 ns/iter; 2.9988x vs baseline; 1.0002x over previous
import jax
import jax.numpy as jnp
from jax import lax
from jax.experimental import pallas as pl
from jax.experimental.pallas import tpu as pltpu

N_GLOBAL = 2048


def kernel(x):
    m, n = x.shape
    BM = 512
    grid = m // BM
    pm = m // 128
    pb = BM // 128
    half = grid // 2
    hrows = pm // 2

    def body(x_ref, out_ref, comm_ref, send_sems, recv_sems):
        i = pl.program_id(0)
        my_x = lax.axis_index("x")
        my_y = lax.axis_index("y")
        peer = (my_x, 1 - my_y)

        def chunk_rdma(c):
            return pltpu.make_async_remote_copy(
                src_ref=comm_ref.at[0, pl.ds(c * hrows, hrows)],
                dst_ref=comm_ref.at[1, pl.ds(c * hrows, hrows)],
                send_sem=send_sems.at[c],
                recv_sem=recv_sems.at[c],
                device_id=peer,
                device_id_type=pl.DeviceIdType.MESH,
            )

        @pl.when(i == 0)
        def _():
            barrier_sem = pltpu.get_barrier_semaphore()
            pl.semaphore_signal(
                barrier_sem,
                inc=1,
                device_id=peer,
                device_id_type=pl.DeviceIdType.MESH,
            )
            pl.semaphore_wait(barrier_sem, 1)

        blk = jnp.sum(x_ref[...], axis=1)
        comm_ref[0, pl.ds(i * pb, pb), :] = blk.reshape(pb, 128)

        @pl.when(i == half - 1)
        def _():
            chunk_rdma(0).start()

        @pl.when(i == grid - 1)
        def _():
            rdma1 = chunk_rdma(1)
            rdma1.start()
            chunk_rdma(0).wait()
            rdma1.wait()
            out_ref[...] = (comm_ref[0] + comm_ref[1]) * (1.0 / N_GLOBAL)

    packed = pl.pallas_call(
        body,
        grid=(grid,),
        in_specs=[pl.BlockSpec((BM, n), lambda i: (i, 0))],
        out_specs=pl.BlockSpec((pm, 128), lambda i: (0, 0)),
        out_shape=jax.ShapeDtypeStruct((pm, 128), jnp.float32),
        scratch_shapes=[
            pltpu.VMEM((2, pm, 128), jnp.float32),
            pltpu.SemaphoreType.DMA((2,)),
            pltpu.SemaphoreType.DMA((2,)),
        ],
        compiler_params=pltpu.CompilerParams(collective_id=0),
    )(x)
    return packed.reshape(m, 1)
